# Initial kernel scaffold; baseline (speedup 1.0000x reference)
import functools

import jax
import jax.numpy as jnp
from jax import lax
from jax.experimental import pallas as pl
from jax.experimental.pallas import tpu as pltpu
from jax.experimental.pallas import tpu_sc as plsc

NC = 2
NS = 16
B = 128
RB = 512


def _cdiv(a, b):
    return (a + b - 1) // b


def _sc_mesh():
    return plsc.VectorSubcoreMesh(core_axis_name="c", subcore_axis_name="s")


def _deg_call(srcd2, n_nodes):
    nchunk_total = srcd2.shape[0]
    nchunk = nchunk_total // (NC * NS)
    acc_rows = NS * _cdiv(n_nodes + 1, NS)
    stripe = acc_rows // NS
    out_stripe = n_nodes // NS

    @functools.partial(
        pl.kernel,
        out_type=jax.ShapeDtypeStruct((NC, n_nodes, 16), jnp.float32),
        mesh=_sc_mesh(),
        scratch_types=[
            pltpu.VMEM((1, B), jnp.int32),
            pltpu.VMEM((B, 16), jnp.float32),
            pltpu.VMEM((B, 16), jnp.float32),
            pltpu.VMEM_SHARED((acc_rows, 16), jnp.float32),
        ],
    )
    def k(srcd_hbm, out_hbm, idxb, onesb, zb, acc):
        c = lax.axis_index("c")
        s = lax.axis_index("s")
        tid = c * NS + s

        @pl.loop(0, B)
        def _(i):
            onesb[pl.ds(i, 1), :] = jnp.ones((1, 16), jnp.float32)
            zb[pl.ds(i, 1), :] = jnp.zeros((1, 16), jnp.float32)

        r0 = s * stripe
        done = 0
        while done < stripe:
            sz = min(B, stripe - done)
            pltpu.sync_copy(zb.at[pl.ds(0, sz)], acc.at[pl.ds(r0 + done, sz)])
            done += sz
        plsc.subcore_barrier()

        base = tid * nchunk

        @pl.loop(0, nchunk)
        def _(i):
            pltpu.sync_copy(srcd_hbm.at[pl.ds(base + i, 1)], idxb)
            pltpu.sync_copy(onesb, acc.at[idxb.at[0]], add=True)

        plsc.subcore_barrier()
        pltpu.sync_copy(acc.at[pl.ds(s * out_stripe, out_stripe)],
                        out_hbm.at[c, pl.ds(s * out_stripe, out_stripe)])

    return k(srcd2)


def _spmm_call(xs, srcg2, dstd2, n_nodes):
    f = xs.shape[1]
    nchunk_total = srcg2.shape[0]
    nchunk = nchunk_total // (NC * NS)
    acc_rows = NS * _cdiv(n_nodes + 1, NS)
    stripe = acc_rows // NS
    out_stripe = n_nodes // NS

    @functools.partial(
        pl.kernel,
        out_type=jax.ShapeDtypeStruct((NC, n_nodes, f), jnp.float32),
        mesh=_sc_mesh(),
        scratch_types=[
            pltpu.VMEM((1, B), jnp.int32),
            pltpu.VMEM((1, B), jnp.int32),
            pltpu.VMEM((B, f), jnp.float32),
            pltpu.VMEM_SHARED((acc_rows, f), jnp.float32),
        ],
    )
    def k(xs_hbm, srcg_hbm, dstd_hbm, out_hbm, srcb, dstb, rows, acc):
        c = lax.axis_index("c")
        s = lax.axis_index("s")
        tid = c * NS + s

        @pl.loop(0, B)
        def _(i):
            for j in range(f // 16):
                rows[pl.ds(i, 1), pl.ds(j * 16, 16)] = jnp.zeros(
                    (1, 16), jnp.float32)

        r0 = s * stripe
        done = 0
        while done < stripe:
            sz = min(B, stripe - done)
            pltpu.sync_copy(rows.at[pl.ds(0, sz)], acc.at[pl.ds(r0 + done, sz)])
            done += sz
        plsc.subcore_barrier()

        base = tid * nchunk

        @pl.loop(0, nchunk)
        def _(i):
            pltpu.sync_copy(srcg_hbm.at[pl.ds(base + i, 1)], srcb)
            pltpu.sync_copy(dstd_hbm.at[pl.ds(base + i, 1)], dstb)
            pltpu.sync_copy(xs_hbm.at[srcb.at[0]], rows)
            pltpu.sync_copy(rows, acc.at[dstb.at[0]], add=True)

        plsc.subcore_barrier()
        pltpu.sync_copy(acc.at[pl.ds(s * out_stripe, out_stripe)],
                        out_hbm.at[c, pl.ds(s * out_stripe, out_stripe)])

    return k(xs, srcg2, dstd2)


def _prep_body(degp_ref, x_ref, dis_ref, u0_ref, u1_ref, u2_ref, u3_ref):
    deg = degp_ref[0] + degp_ref[1]
    dis = jnp.where(deg > 0,
                    lax.rsqrt(jnp.maximum(deg, 1e-12)), 0.0)
    dis_ref[...] = dis
    d1 = dis[:, 0:1]
    u0_ref[...] = d1 * x_ref[0]
    u1_ref[...] = d1 * x_ref[1]
    u2_ref[...] = d1 * x_ref[2]
    u3_ref[...] = d1 * x_ref[3]


def _prep_call(degp, x_seq):
    t, n, f = x_seq.shape
    nb = _cdiv(n, RB)
    row = lambda r: (r, 0)
    out = jax.ShapeDtypeStruct((n, f), jnp.float32)
    return pl.pallas_call(
        _prep_body,
        grid=(nb,),
        in_specs=[
            pl.BlockSpec((2, RB, 16), lambda r: (0, r, 0)),
            pl.BlockSpec((t, RB, f), lambda r: (0, r, 0)),
        ],
        out_specs=[pl.BlockSpec((RB, 16), row)] + [
            pl.BlockSpec((RB, f), row) for _ in range(t)],
        out_shape=[jax.ShapeDtypeStruct((n, 16), jnp.float32)] + [out] * t,
    )(degp, x_seq)


def _gates_body(x_ref, sx_ref, h_ref, sh_ref, dis_ref, w_ref, b_ref,
                z_ref, hr_ref, uhr_ref, txx_ref):
    d1 = dis_ref[:, 0:1]
    txx = -d1 * (sx_ref[0] + sx_ref[1])
    txh = -d1 * (sh_ref[0] + sh_ref[1])
    x = x_ref[...]
    h = h_ref[...]
    acc = jnp.dot(x, w_ref[0:128, :], preferred_element_type=jnp.float32)
    acc += jnp.dot(txx, w_ref[128:256, :], preferred_element_type=jnp.float32)
    acc += jnp.dot(h, w_ref[256:384, :], preferred_element_type=jnp.float32)
    acc += jnp.dot(txh, w_ref[384:512, :], preferred_element_type=jnp.float32)
    zr = jax.nn.sigmoid(acc + b_ref[...])
    z = zr[:, 0:128]
    r = zr[:, 128:256]
    hr = h * r
    z_ref[...] = z
    hr_ref[...] = hr
    uhr_ref[...] = d1 * hr
    txx_ref[...] = txx


def _gates_call(x, sx, h, sh, dis16, wzr, bzr):
    n, f = x.shape
    nb = _cdiv(n, RB)
    row = lambda r: (r, 0)
    nf = jax.ShapeDtypeStruct((n, f), jnp.float32)
    return pl.pallas_call(
        _gates_body,
        grid=(nb,),
        in_specs=[
            pl.BlockSpec((RB, f), row),
            pl.BlockSpec((2, RB, f), lambda r: (0, r, 0)),
            pl.BlockSpec((RB, f), row),
            pl.BlockSpec((2, RB, f), lambda r: (0, r, 0)),
            pl.BlockSpec((RB, 16), row),
            pl.BlockSpec((512, 256), lambda r: (0, 0)),
            pl.BlockSpec((1, 256), lambda r: (0, 0)),
        ],
        out_specs=[pl.BlockSpec((RB, f), row)] * 4,
        out_shape=[nf, nf, nf, nf],
    )(x, sx, h, sh, dis16, wzr, bzr)


def _update_body(x_ref, txx_ref, hr_ref, shr_ref, z_ref, h_ref, dis_ref,
                 w_ref, b_ref, hn_ref, uh_ref):
    d1 = dis_ref[:, 0:1]
    txhr = -d1 * (shr_ref[0] + shr_ref[1])
    acc = jnp.dot(x_ref[...], w_ref[0:128, :],
                  preferred_element_type=jnp.float32)
    acc += jnp.dot(txx_ref[...], w_ref[128:256, :],
                   preferred_element_type=jnp.float32)
    acc += jnp.dot(hr_ref[...], w_ref[256:384, :],
                   preferred_element_type=jnp.float32)
    acc += jnp.dot(txhr, w_ref[384:512, :],
                   preferred_element_type=jnp.float32)
    ht = jnp.tanh(acc + b_ref[...])
    z = z_ref[...]
    hn = z * h_ref[...] + (1.0 - z) * ht
    hn_ref[...] = hn
    uh_ref[...] = d1 * hn


def _update_call(x, txx, hr, shr, z, h, dis16, wh, bh):
    n, f = x.shape
    nb = _cdiv(n, RB)
    row = lambda r: (r, 0)
    nf = jax.ShapeDtypeStruct((n, f), jnp.float32)
    return pl.pallas_call(
        _update_body,
        grid=(nb,),
        in_specs=[
            pl.BlockSpec((RB, f), row),
            pl.BlockSpec((RB, f), row),
            pl.BlockSpec((RB, f), row),
            pl.BlockSpec((2, RB, f), lambda r: (0, r, 0)),
            pl.BlockSpec((RB, f), row),
            pl.BlockSpec((RB, f), row),
            pl.BlockSpec((RB, 16), row),
            pl.BlockSpec((512, 128), lambda r: (0, 0)),
            pl.BlockSpec((1, 128), lambda r: (0, 0)),
        ],
        out_specs=[pl.BlockSpec((RB, f), row)] * 2,
        out_shape=[nf, nf],
    )(x, txx, hr, shr, z, h, dis16, wh, bh)


def _final_body(h_ref, w_ref, b_ref, o_ref):
    o_ref[...] = jnp.dot(h_ref[...], w_ref[...],
                         preferred_element_type=jnp.float32) + b_ref[...]


def _final_call(h, wlin, blin):
    n, f = h.shape
    fo = wlin.shape[1]
    nb = _cdiv(n, RB)
    return pl.pallas_call(
        _final_body,
        grid=(nb,),
        in_specs=[
            pl.BlockSpec((RB, f), lambda r: (r, 0)),
            pl.BlockSpec((f, fo), lambda r: (0, 0)),
            pl.BlockSpec((1, fo), lambda r: (0, 0)),
        ],
        out_specs=pl.BlockSpec((RB, fo), lambda r: (r, 0)),
        out_shape=jax.ShapeDtypeStruct((n, fo), jnp.float32),
    )(h, wlin, blin.reshape(1, fo))


def kernel(X_seq, edge_index, Wxz, bxz, Whz, bhz, Wxr, bxr, Whr, bhr,
           Wxh, bxh, Whh, bhh, Wlin, blin):
    t_steps, n, f = X_seq.shape
    e = edge_index.shape[1]

    src = edge_index[0].astype(jnp.int32)
    dst = edge_index[1].astype(jnp.int32)
    self_loop = src == dst
    srcg = jnp.where(self_loop, 0, src)
    dstd = jnp.where(self_loop, n, dst)
    srcd = jnp.where(self_loop, n, src)

    ept = B * _cdiv(e, NC * NS * B)
    e_pad = NC * NS * ept
    srcg2 = jnp.pad(srcg, (0, e_pad - e)).reshape(e_pad // B, B)
    srcd2 = jnp.pad(srcd, (0, e_pad - e),
                    constant_values=n).reshape(e_pad // B, B)
    dstd2 = jnp.pad(dstd, (0, e_pad - e),
                    constant_values=n).reshape(e_pad // B, B)

    wzr = jnp.concatenate([
        jnp.concatenate([Wxz[0], Wxr[0]], axis=1),
        jnp.concatenate([Wxz[1], Wxr[1]], axis=1),
        jnp.concatenate([Whz[0], Whr[0]], axis=1),
        jnp.concatenate([Whz[1], Whr[1]], axis=1),
    ], axis=0)
    bzr = jnp.concatenate([bxz + bhz, bxr + bhr]).reshape(1, 256)
    wh = jnp.concatenate([Wxh[0], Wxh[1], Whh[0], Whh[1]], axis=0)
    bh = (bxh + bhh).reshape(1, 128)

    degp = _deg_call(srcd2, n)
    prep = _prep_call(degp, X_seq)
    dis16, us = prep[0], prep[1:]

    sx = [_spmm_call(u, srcg2, dstd2, n) for u in us]

    h = jnp.zeros((n, f), jnp.float32)
    sh = jnp.zeros((NC, n, f), jnp.float32)
    for t in range(t_steps):
        z, hr, uhr, txx = _gates_call(X_seq[t], sx[t], h, sh, dis16, wzr, bzr)
        shr = _spmm_call(uhr, srcg2, dstd2, n)
        h, uh = _update_call(X_seq[t], txx, hr, shr, z, h, dis16, wh, bh)
        if t < t_steps - 1:
            sh = _spmm_call(uh, srcg2, dstd2, n)

    return _final_call(h, Wlin, blin)


# speedup vs baseline: 4.4065x; 4.4065x over previous
import functools

import jax
import jax.numpy as jnp
from jax import lax
from jax.experimental import pallas as pl
from jax.experimental.pallas import tpu as pltpu
from jax.experimental.pallas import tpu_sc as plsc

NC = 2
NS = 16
B = 128
RB = 512


def _cdiv(a, b):
    return (a + b - 1) // b


def _sc_mesh():
    return plsc.VectorSubcoreMesh(core_axis_name="c", subcore_axis_name="s")


def _spmm_call(xs, srcg2, dstd2, n_nodes):
    f = xs.shape[1]
    nchunk_total = srcg2.shape[0] // B
    nchunk = nchunk_total // (NC * NS)
    acc_rows = 8 * NS * _cdiv(n_nodes + 1, 8 * NS)
    stripe = acc_rows // NS
    ostripe = 8 * (n_nodes // (8 * NS))
    olast = n_nodes - (NS - 1) * ostripe

    @functools.partial(
        pl.kernel,
        out_type=jax.ShapeDtypeStruct((NC, n_nodes, f), jnp.float32),
        mesh=_sc_mesh(),
        scratch_types=[
            pltpu.VMEM((B,), jnp.int32),
            pltpu.VMEM((B,), jnp.int32),
            pltpu.VMEM((B, f), jnp.float32),
            pltpu.VMEM_SHARED((acc_rows, f), jnp.float32),
        ],
    )
    def k(xs_hbm, srcg_hbm, dstd_hbm, out_hbm, srcb, dstb, rows, acc):
        c = lax.axis_index("c")
        s = lax.axis_index("s")
        tid = c * NS + s

        @pl.loop(0, B)
        def _(i):
            for j in range(f // 16):
                rows[pl.ds(i, 1), pl.ds(j * 16, 16)] = jnp.zeros(
                    (1, 16), jnp.float32)

        r0 = s * stripe
        done = 0
        while done < stripe:
            sz = min(B, stripe - done)
            pltpu.sync_copy(rows.at[pl.ds(0, sz)], acc.at[pl.ds(r0 + done, sz)])
            done += sz
        plsc.subcore_barrier()

        base = tid * nchunk * B

        @pl.loop(0, nchunk)
        def _(i):
            pltpu.sync_copy(srcg_hbm.at[pl.ds(base + i * B, B)], srcb)
            pltpu.sync_copy(dstd_hbm.at[pl.ds(base + i * B, B)], dstb)
            pltpu.sync_copy(xs_hbm.at[srcb], rows)
            pltpu.sync_copy(rows, acc.at[dstb], add=True)

        plsc.subcore_barrier()

        @pl.when(s < NS - 1)
        def _():
            pltpu.sync_copy(acc.at[pl.ds(s * ostripe, ostripe)],
                            out_hbm.at[c, pl.ds(s * ostripe, ostripe)])

        @pl.when(s == NS - 1)
        def _():
            pltpu.sync_copy(acc.at[pl.ds((NS - 1) * ostripe, olast)],
                            out_hbm.at[c, pl.ds((NS - 1) * ostripe, olast)])

    return k(xs, srcg2, dstd2)


def _prep_body(degp_ref, x_ref, dis_ref, u0_ref, u1_ref, u2_ref, u3_ref):
    deg = degp_ref[0, :, 0:16] + degp_ref[1, :, 0:16]
    dis = jnp.where(deg > 0,
                    lax.rsqrt(jnp.maximum(deg, 1e-12)), 0.0)
    dis_ref[...] = dis
    d1 = dis[:, 0:1]
    u0_ref[...] = d1 * x_ref[0]
    u1_ref[...] = d1 * x_ref[1]
    u2_ref[...] = d1 * x_ref[2]
    u3_ref[...] = d1 * x_ref[3]


def _prep_call(degp, x_seq):
    t, n, f = x_seq.shape
    nb = _cdiv(n, RB)
    row = lambda r: (r, 0)
    out = jax.ShapeDtypeStruct((n, f), jnp.float32)
    return pl.pallas_call(
        _prep_body,
        grid=(nb,),
        in_specs=[
            pl.BlockSpec((2, RB, f), lambda r: (0, r, 0)),
            pl.BlockSpec((t, RB, f), lambda r: (0, r, 0)),
        ],
        out_specs=[pl.BlockSpec((RB, 16), row)] + [
            pl.BlockSpec((RB, f), row) for _ in range(t)],
        out_shape=[jax.ShapeDtypeStruct((n, 16), jnp.float32)] + [out] * t,
    )(degp, x_seq)


def _gates_body(x_ref, sx_ref, h_ref, sh_ref, dis_ref, w_ref, b_ref,
                z_ref, hr_ref, uhr_ref, txx_ref):
    d1 = dis_ref[:, 0:1]
    txx = -d1 * (sx_ref[0] + sx_ref[1])
    txh = -d1 * (sh_ref[0] + sh_ref[1])
    x = x_ref[...]
    h = h_ref[...]
    acc = jnp.dot(x, w_ref[0:128, :], preferred_element_type=jnp.float32)
    acc += jnp.dot(txx, w_ref[128:256, :], preferred_element_type=jnp.float32)
    acc += jnp.dot(h, w_ref[256:384, :], preferred_element_type=jnp.float32)
    acc += jnp.dot(txh, w_ref[384:512, :], preferred_element_type=jnp.float32)
    zr = jax.nn.sigmoid(acc + b_ref[...])
    z = zr[:, 0:128]
    r = zr[:, 128:256]
    hr = h * r
    z_ref[...] = z
    hr_ref[...] = hr
    uhr_ref[...] = d1 * hr
    txx_ref[...] = txx


def _gates_call(x, sx, h, sh, dis16, wzr, bzr):
    n, f = x.shape
    nb = _cdiv(n, RB)
    row = lambda r: (r, 0)
    nf = jax.ShapeDtypeStruct((n, f), jnp.float32)
    return pl.pallas_call(
        _gates_body,
        grid=(nb,),
        in_specs=[
            pl.BlockSpec((RB, f), row),
            pl.BlockSpec((2, RB, f), lambda r: (0, r, 0)),
            pl.BlockSpec((RB, f), row),
            pl.BlockSpec((2, RB, f), lambda r: (0, r, 0)),
            pl.BlockSpec((RB, 16), row),
            pl.BlockSpec((512, 256), lambda r: (0, 0)),
            pl.BlockSpec((1, 256), lambda r: (0, 0)),
        ],
        out_specs=[pl.BlockSpec((RB, f), row)] * 4,
        out_shape=[nf, nf, nf, nf],
    )(x, sx, h, sh, dis16, wzr, bzr)


def _update_body(x_ref, txx_ref, hr_ref, shr_ref, z_ref, h_ref, dis_ref,
                 w_ref, b_ref, hn_ref, uh_ref):
    d1 = dis_ref[:, 0:1]
    txhr = -d1 * (shr_ref[0] + shr_ref[1])
    acc = jnp.dot(x_ref[...], w_ref[0:128, :],
                  preferred_element_type=jnp.float32)
    acc += jnp.dot(txx_ref[...], w_ref[128:256, :],
                   preferred_element_type=jnp.float32)
    acc += jnp.dot(hr_ref[...], w_ref[256:384, :],
                   preferred_element_type=jnp.float32)
    acc += jnp.dot(txhr, w_ref[384:512, :],
                   preferred_element_type=jnp.float32)
    ht = jnp.tanh(acc + b_ref[...])
    z = z_ref[...]
    hn = z * h_ref[...] + (1.0 - z) * ht
    hn_ref[...] = hn
    uh_ref[...] = d1 * hn


def _update_call(x, txx, hr, shr, z, h, dis16, wh, bh):
    n, f = x.shape
    nb = _cdiv(n, RB)
    row = lambda r: (r, 0)
    nf = jax.ShapeDtypeStruct((n, f), jnp.float32)
    return pl.pallas_call(
        _update_body,
        grid=(nb,),
        in_specs=[
            pl.BlockSpec((RB, f), row),
            pl.BlockSpec((RB, f), row),
            pl.BlockSpec((RB, f), row),
            pl.BlockSpec((2, RB, f), lambda r: (0, r, 0)),
            pl.BlockSpec((RB, f), row),
            pl.BlockSpec((RB, f), row),
            pl.BlockSpec((RB, 16), row),
            pl.BlockSpec((512, 128), lambda r: (0, 0)),
            pl.BlockSpec((1, 128), lambda r: (0, 0)),
        ],
        out_specs=[pl.BlockSpec((RB, f), row)] * 2,
        out_shape=[nf, nf],
    )(x, txx, hr, shr, z, h, dis16, wh, bh)


def _final_body(h_ref, w_ref, b_ref, o_ref):
    o_ref[...] = jnp.dot(h_ref[...], w_ref[...],
                         preferred_element_type=jnp.float32) + b_ref[...]


def _final_call(h, wlin, blin):
    n, f = h.shape
    fo = wlin.shape[1]
    nb = _cdiv(n, RB)
    return pl.pallas_call(
        _final_body,
        grid=(nb,),
        in_specs=[
            pl.BlockSpec((RB, f), lambda r: (r, 0)),
            pl.BlockSpec((f, fo), lambda r: (0, 0)),
            pl.BlockSpec((1, fo), lambda r: (0, 0)),
        ],
        out_specs=pl.BlockSpec((RB, fo), lambda r: (r, 0)),
        out_shape=jax.ShapeDtypeStruct((n, fo), jnp.float32),
    )(h, wlin, blin.reshape(1, fo))


def kernel(X_seq, edge_index, Wxz, bxz, Whz, bhz, Wxr, bxr, Whr, bhr,
           Wxh, bxh, Whh, bhh, Wlin, blin):
    t_steps, n, f = X_seq.shape
    e = edge_index.shape[1]

    src = edge_index[0].astype(jnp.int32)
    dst = edge_index[1].astype(jnp.int32)
    self_loop = src == dst
    srcg = jnp.where(self_loop, 0, src)
    dstd = jnp.where(self_loop, n, dst)
    srcd = jnp.where(self_loop, n, src)

    ept = B * _cdiv(e, NC * NS * B)
    e_pad = NC * NS * ept
    srcg2 = jnp.pad(srcg, (0, e_pad - e))
    srcd2 = jnp.pad(srcd, (0, e_pad - e), constant_values=n)
    dstd2 = jnp.pad(dstd, (0, e_pad - e), constant_values=n)

    wzr = jnp.concatenate([
        jnp.concatenate([Wxz[0], Wxr[0]], axis=1),
        jnp.concatenate([Wxz[1], Wxr[1]], axis=1),
        jnp.concatenate([Whz[0], Whr[0]], axis=1),
        jnp.concatenate([Whz[1], Whr[1]], axis=1),
    ], axis=0)
    bzr = jnp.concatenate([bxz + bhz, bxr + bhr]).reshape(1, 256)
    wh = jnp.concatenate([Wxh[0], Wxh[1], Whh[0], Whh[1]], axis=0)
    bh = (bxh + bhh).reshape(1, 128)

    degp = _spmm_call(jnp.ones((n, f), jnp.float32), srcg2, srcd2, n)
    prep = _prep_call(degp, X_seq)
    dis16, us = prep[0], prep[1:]

    sx = [_spmm_call(u, srcg2, dstd2, n) for u in us]

    h = jnp.zeros((n, f), jnp.float32)
    sh = jnp.zeros((NC, n, f), jnp.float32)
    for t in range(t_steps):
        z, hr, uhr, txx = _gates_call(X_seq[t], sx[t], h, sh, dis16, wzr, bzr)
        shr = _spmm_call(uhr, srcg2, dstd2, n)
        h, uh = _update_call(X_seq[t], txx, hr, shr, z, h, dis16, wh, bh)
        if t < t_steps - 1:
            sh = _spmm_call(uh, srcg2, dstd2, n)

    return _final_call(h, Wlin, blin)
